# baseline (speedup 1.0000x reference)
import functools

import jax
import jax.numpy as jnp
from jax import lax
from jax.experimental import pallas as pl
from jax.experimental.pallas import tpu as pltpu
from jax.experimental.pallas import tpu_sc as plsc


def _make_sc_kernel(batch, seqlen, dim):
    info = plsc.get_sparse_core_info()
    nc, ns, lanes = info.num_cores, info.num_subcores, info.num_lanes
    nw = nc * ns
    assert seqlen % nw == 0
    chunk = (seqlen // nw) * dim

    mesh = plsc.VectorSubcoreMesh(core_axis_name="c", subcore_axis_name="s")

    @functools.partial(
        pl.kernel,
        mesh=mesh,
        out_type=jax.ShapeDtypeStruct((batch * seqlen * dim,), jnp.float32),
        scratch_types=[
            pltpu.VMEM((chunk,), jnp.float32),
            pltpu.VMEM((chunk,), jnp.float32),
        ],
    )
    def sc_add(x_hbm, pos_hbm, out_hbm, pos_v, buf_v):
        wid = lax.axis_index("s") * nc + lax.axis_index("c")
        pos_base = wid * chunk
        pltpu.sync_copy(pos_hbm.at[pl.ds(pos_base, chunk)], pos_v)
        for b in range(batch):
            base = b * seqlen * dim + pos_base
            pltpu.sync_copy(x_hbm.at[pl.ds(base, chunk)], buf_v)

            def body(i, carry):
                sl = pl.ds(i * lanes, lanes)
                buf_v[sl] = buf_v[sl] + pos_v[sl]
                return carry

            lax.fori_loop(0, chunk // lanes, body, 0)
            pltpu.sync_copy(buf_v, out_hbm.at[pl.ds(base, chunk)])

    return sc_add


def kernel(inputs, pos_table):
    batch, seqlen, dim = inputs.shape
    sc_add = _make_sc_kernel(batch, seqlen, dim)
    out = sc_add(inputs.reshape(-1), pos_table[:seqlen].reshape(-1))
    return out.reshape(batch, seqlen, dim)


# speedup vs baseline: 1.5283x; 1.5283x over previous
import functools

import jax
import jax.numpy as jnp
from jax import lax
from jax.experimental import pallas as pl
from jax.experimental.pallas import tpu as pltpu
from jax.experimental.pallas import tpu_sc as plsc

_NBUF = 3
_NSUB = 2
_UNROLL = 8


def _make_sc_kernel(batch, seqlen, dim):
    info = plsc.get_sparse_core_info()
    nc, ns, lanes = info.num_cores, info.num_subcores, info.num_lanes
    nw = nc * ns
    assert seqlen % nw == 0
    chunk = (seqlen // nw) * dim
    sub = chunk // _NSUB
    ntot = batch * _NSUB
    assert sub % (lanes * _UNROLL) == 0

    mesh = plsc.VectorSubcoreMesh(core_axis_name="c", subcore_axis_name="s")

    @functools.partial(
        pl.kernel,
        mesh=mesh,
        out_type=jax.ShapeDtypeStruct((batch * seqlen * dim,), jnp.float32),
        scratch_types=[pltpu.VMEM((chunk,), jnp.float32)]
        + [pltpu.VMEM((sub,), jnp.float32) for _ in range(_NBUF)]
        + [pltpu.SemaphoreType.DMA for _ in range(2 * _NBUF)],
    )
    def sc_add(x_hbm, pos_hbm, out_hbm, pos_v, b0, b1, b2, i0, i1, i2, o0, o1, o2):
        bufs = [b0, b1, b2]
        in_sems = [i0, i1, i2]
        out_sems = [o0, o1, o2]
        wid = lax.axis_index("s") * nc + lax.axis_index("c")
        pos_base = wid * chunk
        pltpu.sync_copy(pos_hbm.at[pl.ds(pos_base, chunk)], pos_v)

        def src(k):
            b, j = divmod(k, _NSUB)
            return b * seqlen * dim + pos_base + j * sub

        in_cp = [None] * ntot
        out_cp = [None] * ntot
        in_cp[0] = pltpu.async_copy(x_hbm.at[pl.ds(src(0), sub)], bufs[0], in_sems[0])
        for k in range(ntot):
            bi = k % _NBUF
            if k + 1 < ntot:
                nbi = (k + 1) % _NBUF
                if k + 1 >= _NBUF:
                    out_cp[k + 1 - _NBUF].wait()
                in_cp[k + 1] = pltpu.async_copy(
                    x_hbm.at[pl.ds(src(k + 1), sub)], bufs[nbi], in_sems[nbi]
                )
            in_cp[k].wait()
            pbase = (k % _NSUB) * sub
            buf = bufs[bi]

            def body(i, carry, _buf=buf, _pbase=pbase):
                v0 = i * (lanes * _UNROLL)
                for u in range(_UNROLL):
                    off = v0 + u * lanes
                    plsc.addupdate(
                        _buf.at[pl.ds(off, lanes)],
                        pos_v[pl.ds(_pbase + off, lanes)],
                    )
                return carry

            lax.fori_loop(0, sub // (lanes * _UNROLL), body, 0)
            out_cp[k] = pltpu.async_copy(
                bufs[bi], out_hbm.at[pl.ds(src(k), sub)], out_sems[bi]
            )
        for k in range(max(0, ntot - _NBUF), ntot):
            out_cp[k].wait()

    return sc_add


def kernel(inputs, pos_table):
    batch, seqlen, dim = inputs.shape
    sc_add = _make_sc_kernel(batch, seqlen, dim)
    out = sc_add(inputs.reshape(-1), pos_table[:seqlen].reshape(-1))
    return out.reshape(batch, seqlen, dim)


# speedup vs baseline: 4.6733x; 3.0577x over previous
import jax
import jax.numpy as jnp
from jax.experimental import pallas as pl


def _add_body(x_ref, p_ref, o_ref):
    o_ref[...] = x_ref[...] + p_ref[...]


def kernel(inputs, pos_table):
    batch, seqlen, dim = inputs.shape
    bs = 256
    grid = (seqlen // bs, batch)
    return pl.pallas_call(
        _add_body,
        grid=grid,
        in_specs=[
            pl.BlockSpec((1, bs, dim), lambda s, b: (b, s, 0)),
            pl.BlockSpec((bs, dim), lambda s, b: (s, 0)),
        ],
        out_specs=pl.BlockSpec((1, bs, dim), lambda s, b: (b, s, 0)),
        out_shape=jax.ShapeDtypeStruct(inputs.shape, inputs.dtype),
    )(inputs, pos_table[:seqlen])


# speedup vs baseline: 7.5428x; 1.6140x over previous
import jax
import jax.numpy as jnp
from jax.experimental import pallas as pl


def _add_body(x_ref, p_ref, o_ref):
    o_ref[...] = x_ref[...] + p_ref[...]


def kernel(inputs, pos_table):
    batch, seqlen, dim = inputs.shape
    bs = 1024
    grid = (seqlen // bs, batch)
    return pl.pallas_call(
        _add_body,
        grid=grid,
        in_specs=[
            pl.BlockSpec((1, bs, dim), lambda s, b: (b, s, 0)),
            pl.BlockSpec((bs, dim), lambda s, b: (s, 0)),
        ],
        out_specs=pl.BlockSpec((1, bs, dim), lambda s, b: (b, s, 0)),
        out_shape=jax.ShapeDtypeStruct(inputs.shape, inputs.dtype),
    )(inputs, pos_table[:seqlen])


# speedup vs baseline: 8.2109x; 1.0886x over previous
import jax
import jax.numpy as jnp
from jax.experimental import pallas as pl


def _add_body(x_ref, p_ref, o_ref):
    o_ref[...] = x_ref[...] + p_ref[...]


def kernel(inputs, pos_table):
    batch, seqlen, dim = inputs.shape
    bs = 2048
    grid = (seqlen // bs, batch)
    return pl.pallas_call(
        _add_body,
        grid=grid,
        in_specs=[
            pl.BlockSpec((1, bs, dim), lambda s, b: (b, s, 0)),
            pl.BlockSpec((bs, dim), lambda s, b: (s, 0)),
        ],
        out_specs=pl.BlockSpec((1, bs, dim), lambda s, b: (b, s, 0)),
        out_shape=jax.ShapeDtypeStruct(inputs.shape, inputs.dtype),
    )(inputs, pos_table[:seqlen])
